# baseline (speedup 1.0000x reference)
import functools

import jax
import jax.numpy as jnp
from jax import lax
from jax.experimental import pallas as pl
from jax.experimental.pallas import tpu as pltpu
from jax.experimental.pallas import tpu_sc as plsc

_B = 16384
_D = 32
_PK = 4
_NC = 2
_NS = 16
_NW = _NC * _NS
_BPW = _B // _NW
_CH = 64


def _fpmc_body(uid_hbm, lid_hbm, pid_hbm, ui_hbm, iu_hbm, il_hbm, li_hbm,
               out_hbm,
               idx_u, idx_l, idx_p, mu_v, ml_v, mp_v, ui_v, iu_v, il_v, li_v,
               out_v,
               sem0, sem1, sem2, sem3):
    wid = lax.axis_index("s") * _NC + lax.axis_index("c")
    base = wid * _BPW

    pltpu.sync_copy(uid_hbm.at[pl.ds(base, _BPW)], idx_u)
    pltpu.sync_copy(lid_hbm.at[pl.ds(base, _BPW)], idx_l)
    pltpu.sync_copy(pid_hbm.at[pl.ds(base, _BPW)], idx_p)

    for ch in range(_BPW // 16):
        c = pl.ds(ch * 16, 16)
        mu_v[c] = (idx_u[c] & 3) * _D
        ml_v[c] = (idx_l[c] & 3) * _D
        mp_v[c] = (idx_p[c] & 3) * _D
        idx_u[c] = jax.lax.shift_right_logical(idx_u[c], 2)
        idx_l[c] = jax.lax.shift_right_logical(idx_l[c], 2)
        idx_p[c] = jax.lax.shift_right_logical(idx_p[c], 2)

    lane = lax.broadcasted_iota(jnp.int32, (16,), 0)

    def body(g, carry):
        gc = pl.ds(g * _CH, _CH)
        cu = pltpu.async_copy(ui_hbm.at[idx_u.at[gc]], ui_v, sem0)
        ci = pltpu.async_copy(iu_hbm.at[idx_p.at[gc]], iu_v, sem1)
        cm = pltpu.async_copy(li_hbm.at[idx_l.at[gc]], li_v, sem2)
        cn = pltpu.async_copy(il_hbm.at[idx_p.at[gc]], il_v, sem3)
        cu.wait()
        ci.wait()
        cm.wait()
        cn.wait()
        for gg in range(_CH // 16):
            row = gg * 16 + lane
            bc = pl.ds(g * _CH + gg * 16, 16)
            mu = mu_v[bc]
            ml = ml_v[bc]
            mp = mp_v[bc]
            acc0 = jnp.zeros((16,), jnp.float32)
            acc1 = jnp.zeros((16,), jnp.float32)
            for d in range(_D):
                u = plsc.load_gather(ui_v, [row, mu + d])
                i = plsc.load_gather(iu_v, [row, mp + d])
                m = plsc.load_gather(li_v, [row, ml + d])
                n = plsc.load_gather(il_v, [row, mp + d])
                acc0 = acc0 + u * i
                acc1 = acc1 + m * n
            out_v[bc] = acc0 + acc1
        return carry

    lax.fori_loop(0, _BPW // _CH, body, 0)
    pltpu.sync_copy(out_v, out_hbm.at[pl.ds(base, _BPW)])


@jax.jit
def _fpmc(user_ids, last_items, pre_items, UI, IU, IL, LI):
    UIr = jnp.reshape(UI, (UI.shape[0] // _PK, _D * _PK))
    IUr = jnp.reshape(IU, (IU.shape[0] // _PK, _D * _PK))
    ILr = jnp.reshape(IL, (IL.shape[0] // _PK, _D * _PK))
    LIr = jnp.reshape(LI, (LI.shape[0] // _PK, _D * _PK))
    mesh = plsc.VectorSubcoreMesh(core_axis_name="c", subcore_axis_name="s")
    run = pl.kernel(
        _fpmc_body,
        out_type=jax.ShapeDtypeStruct((_B,), jnp.float32),
        mesh=mesh,
        compiler_params=pltpu.CompilerParams(needs_layout_passes=False),
        scratch_types=[
            pltpu.VMEM((_BPW,), jnp.int32),
            pltpu.VMEM((_BPW,), jnp.int32),
            pltpu.VMEM((_BPW,), jnp.int32),
            pltpu.VMEM((_BPW,), jnp.int32),
            pltpu.VMEM((_BPW,), jnp.int32),
            pltpu.VMEM((_BPW,), jnp.int32),
            pltpu.VMEM((_CH, _D * _PK), jnp.float32),
            pltpu.VMEM((_CH, _D * _PK), jnp.float32),
            pltpu.VMEM((_CH, _D * _PK), jnp.float32),
            pltpu.VMEM((_CH, _D * _PK), jnp.float32),
            pltpu.VMEM((_BPW,), jnp.float32),
            pltpu.SemaphoreType.DMA,
            pltpu.SemaphoreType.DMA,
            pltpu.SemaphoreType.DMA,
            pltpu.SemaphoreType.DMA,
        ],
    )
    return run(user_ids, last_items, pre_items, UIr, IUr, ILr, LIr)


def kernel(user_ids, last_items, pre_items, UI, IU, IL, LI):
    return _fpmc(user_ids.astype(jnp.int32), last_items.astype(jnp.int32),
                 pre_items.astype(jnp.int32), UI, IU, IL, LI)


# speedup vs baseline: 1.0325x; 1.0325x over previous
import functools

import jax
import jax.numpy as jnp
from jax import lax
from jax.experimental import pallas as pl
from jax.experimental.pallas import tpu as pltpu
from jax.experimental.pallas import tpu_sc as plsc

_B = 16384
_D = 32
_NC = 2
_NS = 16
_NW = _NC * _NS
_BPW = _B // _NW


def _fpmc_body(uid_hbm, lid_hbm, pid_hbm, ui_hbm, iu_hbm, il_hbm, li_hbm,
               out_hbm,
               idx_u, idx_l, idx_p, ui_v, iu_v, il_v, li_v, tr_v, out_v,
               sem0, sem1, sem2, sem3):
    wid = lax.axis_index("s") * _NC + lax.axis_index("c")
    base = wid * _BPW

    pltpu.sync_copy(uid_hbm.at[pl.ds(base, _BPW)], idx_u)
    pltpu.sync_copy(lid_hbm.at[pl.ds(base, _BPW)], idx_l)
    pltpu.sync_copy(pid_hbm.at[pl.ds(base, _BPW)], idx_p)

    cu = pltpu.async_copy(ui_hbm.at[idx_u], ui_v, sem0)
    ci = pltpu.async_copy(iu_hbm.at[idx_p], iu_v, sem1)
    cl = pltpu.async_copy(il_hbm.at[idx_p], il_v, sem2)
    cm = pltpu.async_copy(li_hbm.at[idx_l], li_v, sem3)
    cu.wait()
    ci.wait()
    cl.wait()
    cm.wait()

    lane = lax.broadcasted_iota(jnp.int32, (16,), 0)

    def body(g, carry):
        for j in range(16):
            b = g * 16 + j
            p = ui_v[b, pl.ds(0, 16)] * iu_v[b, pl.ds(0, 16)]
            p = p + ui_v[b, pl.ds(16, 16)] * iu_v[b, pl.ds(16, 16)]
            p = p + li_v[b, pl.ds(0, 16)] * il_v[b, pl.ds(0, 16)]
            p = p + li_v[b, pl.ds(16, 16)] * il_v[b, pl.ds(16, 16)]
            plsc.store_scatter(tr_v, [lane * 16 + j], p)
        acc = tr_v[pl.ds(0, 16)]
        for i in range(1, 16):
            acc = acc + tr_v[pl.ds(i * 16, 16)]
        out_v[pl.ds(g * 16, 16)] = acc
        return carry

    lax.fori_loop(0, _BPW // 16, body, 0)
    pltpu.sync_copy(out_v, out_hbm.at[pl.ds(base, _BPW)])


@jax.jit
def _fpmc(user_ids, last_items, pre_items, UI, IU, IL, LI):
    mesh = plsc.VectorSubcoreMesh(core_axis_name="c", subcore_axis_name="s")
    run = pl.kernel(
        _fpmc_body,
        out_type=jax.ShapeDtypeStruct((_B,), jnp.float32),
        mesh=mesh,
        compiler_params=pltpu.CompilerParams(
            needs_layout_passes=False, use_tc_tiling_on_sc=False),
        scratch_types=[
            pltpu.VMEM((_BPW,), jnp.int32),
            pltpu.VMEM((_BPW,), jnp.int32),
            pltpu.VMEM((_BPW,), jnp.int32),
            pltpu.VMEM((_BPW, _D), jnp.float32),
            pltpu.VMEM((_BPW, _D), jnp.float32),
            pltpu.VMEM((_BPW, _D), jnp.float32),
            pltpu.VMEM((_BPW, _D), jnp.float32),
            pltpu.VMEM((256,), jnp.float32),
            pltpu.VMEM((_BPW,), jnp.float32),
            pltpu.SemaphoreType.DMA,
            pltpu.SemaphoreType.DMA,
            pltpu.SemaphoreType.DMA,
            pltpu.SemaphoreType.DMA,
        ],
    )
    return run(user_ids, last_items, pre_items, UI, IU, IL, LI)


def kernel(user_ids, last_items, pre_items, UI, IU, IL, LI):
    return _fpmc(user_ids.astype(jnp.int32), last_items.astype(jnp.int32),
                 pre_items.astype(jnp.int32), UI, IU, IL, LI)
